# Initial kernel scaffold; baseline (speedup 1.0000x reference)
import jax
import jax.numpy as jnp
from jax.experimental import pallas as pl


def kernel(x, edge_index, Wg, bg, W1, b1, W2, b2):
    raise NotImplementedError("write your pallas kernel here")


# speedup vs baseline: 16.4291x; 16.4291x over previous
import functools

import jax
import jax.numpy as jnp
from jax import lax
from jax.experimental import pallas as pl
from jax.experimental.pallas import tpu as pltpu
from jax.experimental.pallas import tpu_sc as plsc

NC = 2
NS = 16
NW = NC * NS
L = 128


def _deg_body(dst_hbm, zeros_hbm, out_hbm, dst_v, ones_v, deg_sh, K, SLAB):
    c = lax.axis_index("c")
    s = lax.axis_index("s")
    w = c * NS + s
    pltpu.sync_copy(zeros_hbm, deg_sh.at[pl.ds(s * SLAB, SLAB)])
    pltpu.sync_copy(dst_hbm.at[w], dst_v)
    for k in range(L // 16):
        ones_v[pl.ds(k * 16, 16)] = jnp.ones((16,), jnp.float32)
    plsc.subcore_barrier()

    def body(j, carry):
        pltpu.sync_copy(ones_v, deg_sh.at[dst_v.at[j]], add=True)
        return carry

    lax.fori_loop(0, K, body, 0)
    plsc.subcore_barrier()
    pltpu.sync_copy(deg_sh.at[pl.ds(s * SLAB, SLAB)],
                    out_hbm.at[c, pl.ds(s * SLAB, SLAB)])


def _edge_body(hp_hbm, src_hbm, dst_hbm, zeros_hbm, out_hbm,
               src_v, dst_v, rows_v, acc_sh, K, SLAB):
    c = lax.axis_index("c")
    s = lax.axis_index("s")
    w = c * NS + s
    pltpu.sync_copy(zeros_hbm, acc_sh.at[pl.ds(s * SLAB, SLAB)])
    pltpu.sync_copy(src_hbm.at[w], src_v)
    pltpu.sync_copy(dst_hbm.at[w], dst_v)
    plsc.subcore_barrier()

    def body(j, carry):
        pltpu.sync_copy(hp_hbm.at[src_v.at[j]], rows_v)
        pltpu.sync_copy(rows_v, acc_sh.at[dst_v.at[j]], add=True)
        return carry

    lax.fori_loop(0, K, body, 0)
    plsc.subcore_barrier()
    pltpu.sync_copy(acc_sh.at[pl.ds(s * SLAB, SLAB)],
                    out_hbm.at[c, pl.ds(s * SLAB, SLAB)])


def _mm_scale_body(x_ref, wg_ref, degs_ref, hp_ref):
    h = jnp.dot(x_ref[...], wg_ref[...], preferred_element_type=jnp.float32)
    d = degs_ref[:, 0:1] + degs_ref[:, 1:2] + 1.0
    hp_ref[...] = h * lax.rsqrt(d)


def _tail_body(acc_ref, hp_ref, degs_ref, bg_ref, w1_ref, b1_ref, w2_ref,
               b2_ref, y_ref):
    agg = acc_ref[0] + acc_ref[1] + hp_ref[...]
    dis = lax.rsqrt(degs_ref[:, 0:1] + degs_ref[:, 1:2] + 1.0)
    m = jnp.maximum(agg * dis + bg_ref[...], 0.0)
    h2 = jnp.maximum(
        jnp.dot(m, w1_ref[...], preferred_element_type=jnp.float32)
        + b1_ref[...], 0.0)
    y_ref[...] = (jnp.dot(h2, w2_ref[...], preferred_element_type=jnp.float32)
                  + b2_ref[...])


def kernel(x, edge_index, Wg, bg, W1, b1, W2, b2):
    N, F = x.shape
    E = edge_index.shape[1]
    K = -(-E // (NW * L))
    EP = NW * K * L
    NP = ((N + NS * 8 - 1) // (NS * 8)) * (NS * 8) + NS * 8
    SLAB = NP // NS

    src = edge_index[0].astype(jnp.int32)
    dst = edge_index[1].astype(jnp.int32)
    pad = EP - E
    src_p = jnp.concatenate([src, jnp.zeros((pad,), jnp.int32)]).reshape(NW, K, L)
    dst_p = jnp.concatenate([dst, jnp.full((pad,), N, jnp.int32)]).reshape(NW, K, L)
    zeros1 = jnp.zeros((SLAB,), jnp.float32)
    zeros2 = jnp.zeros((SLAB, F), jnp.float32)

    mesh = plsc.VectorSubcoreMesh(core_axis_name="c", subcore_axis_name="s")

    deg_call = functools.partial(
        pl.kernel,
        functools.partial(_deg_body, K=K, SLAB=SLAB),
        out_type=jax.ShapeDtypeStruct((NC, NP), jnp.float32),
        mesh=mesh,
        scratch_types=[
            pltpu.VMEM((K, L), jnp.int32),
            pltpu.VMEM((L,), jnp.float32),
            pltpu.VMEM_SHARED((NP,), jnp.float32),
        ],
    )()
    degs = deg_call(dst_p, zeros1)
    degs_t = degs.T

    RB = 1000
    grid = N // RB
    hp = pl.pallas_call(
        _mm_scale_body,
        grid=(grid,),
        in_specs=[
            pl.BlockSpec((RB, F), lambda i: (i, 0)),
            pl.BlockSpec((F, F), lambda i: (0, 0)),
            pl.BlockSpec((RB, NC), lambda i: (i, 0)),
        ],
        out_specs=pl.BlockSpec((RB, F), lambda i: (i, 0)),
        out_shape=jax.ShapeDtypeStruct((N, F), jnp.float32),
    )(x, Wg, degs_t[:N])

    edge_call = functools.partial(
        pl.kernel,
        functools.partial(_edge_body, K=K, SLAB=SLAB),
        out_type=jax.ShapeDtypeStruct((NC, NP, F), jnp.float32),
        mesh=mesh,
        scratch_types=[
            pltpu.VMEM((K, L), jnp.int32),
            pltpu.VMEM((K, L), jnp.int32),
            pltpu.VMEM((L, F), jnp.float32),
            pltpu.VMEM_SHARED((NP, F), jnp.float32),
        ],
    )()
    acc = edge_call(hp, src_p, dst_p, zeros2)

    y = pl.pallas_call(
        _tail_body,
        grid=(grid,),
        in_specs=[
            pl.BlockSpec((NC, RB, F), lambda i: (0, i, 0)),
            pl.BlockSpec((RB, F), lambda i: (i, 0)),
            pl.BlockSpec((RB, NC), lambda i: (i, 0)),
            pl.BlockSpec((1, F), lambda i: (0, 0)),
            pl.BlockSpec((F, F), lambda i: (0, 0)),
            pl.BlockSpec((1, F), lambda i: (0, 0)),
            pl.BlockSpec((F, 1), lambda i: (0, 0)),
            pl.BlockSpec((1, 1), lambda i: (0, 0)),
        ],
        out_specs=pl.BlockSpec((RB, 1), lambda i: (i, 0)),
        out_shape=jax.ShapeDtypeStruct((N, 1), jnp.float32),
    )(acc[:, :N], hp, degs_t[:N], bg.reshape(1, F), W1, b1.reshape(1, F),
      W2, b2.reshape(1, 1))
    return y


# speedup vs baseline: 39.4197x; 2.3994x over previous
import functools

import jax
import jax.numpy as jnp
from jax import lax
from jax.experimental import pallas as pl
from jax.experimental.pallas import tpu as pltpu
from jax.experimental.pallas import tpu_sc as plsc

NC = 2
NS = 16
NW = NC * NS
L = 80
_D = 4
_DI = 8
_AI = 4
_AG = 2
_S = 2


def _deg_body(idx_hbm, zeros_hbm, out_hbm, dst_v, ones_v, deg_sh, K, SLAB):
    c = lax.axis_index("c")
    s = lax.axis_index("s")
    w = c * NS + s
    pltpu.sync_copy(zeros_hbm, deg_sh.at[pl.ds(s * SLAB, SLAB)])
    pltpu.sync_copy(idx_hbm.at[w], dst_v)
    for k in range(L // 16):
        ones_v[pl.ds(k * 16, 16)] = jnp.ones((16,), jnp.float32)
    plsc.subcore_barrier()

    def body(j, carry):
        pltpu.sync_copy(ones_v, deg_sh.at[dst_v.at[j, 1]], add=True)
        return carry

    lax.fori_loop(0, K, body, 0)
    plsc.subcore_barrier()
    pltpu.sync_copy(deg_sh.at[pl.ds(s * SLAB, SLAB)],
                    out_hbm.at[c, pl.ds(s * SLAB, SLAB)])


def _edge_body(hp_hbm, idx_hbm, zeros_hbm, out_hbm,
               idx_v, r0, r1, r2, r3, acc_sh,
               i0, i1, i2, i3, i4, i5, i6, i7,
               g0, g1, g2, g3, s0, s1, s2, s3, K, SLAB):
    c = lax.axis_index("c")
    s = lax.axis_index("s")
    w = c * NS + s
    rows = (r0, r1, r2, r3)
    isem = (i0, i1, i2, i3, i4, i5, i6, i7)
    gsem = (g0, g1, g2, g3)
    ssem = (s0, s1, s2, s3)
    pltpu.sync_copy(zeros_hbm, acc_sh.at[pl.ds(s * SLAB, SLAB)])
    plsc.subcore_barrier()

    for j in range(min(_AI, K)):
        pltpu.async_copy(idx_hbm.at[w, j], idx_v.at[j % _DI], isem[j % _DI])
    for j in range(min(_AG, K)):
        si = j % _DI
        pltpu.make_async_copy(idx_hbm.at[w, j], idx_v.at[si], isem[si]).wait()
        pltpu.async_copy(hp_hbm.at[idx_v.at[si, 0]], rows[j % _D],
                         gsem[j % _D])

    def body(j, carry):
        for u in range(8):

            @pl.when(j % 8 == u)
            def _(u=u):
                p = u % _D

                @pl.when(j >= _S)
                def _():
                    q = (u - _S) % _D
                    qi = (u - _S) % _DI
                    pltpu.make_async_copy(
                        rows[q], acc_sh.at[idx_v.at[qi, 1]], ssem[q]).wait()

                @pl.when(j + _AI <= K - 1)
                def _():
                    si = (u + _AI) % _DI
                    pltpu.async_copy(idx_hbm.at[w, j + _AI], idx_v.at[si],
                                     isem[si])

                @pl.when(j + _AG <= K - 1)
                def _():
                    sg = (u + _AG) % _DI
                    rq = (u + _AG) % _D
                    pltpu.make_async_copy(idx_hbm.at[w, j + _AG],
                                          idx_v.at[sg], isem[sg]).wait()
                    pltpu.async_copy(hp_hbm.at[idx_v.at[sg, 0]], rows[rq],
                                     gsem[rq])

                ui = u % _DI
                pltpu.make_async_copy(hp_hbm.at[idx_v.at[ui, 0]], rows[p],
                                      gsem[p]).wait()
                pltpu.async_copy(rows[p], acc_sh.at[idx_v.at[ui, 1]], ssem[p],
                                 add=True)

        return carry

    lax.fori_loop(0, K, body, 0)
    for j in range(max(K - _S, 0), K):
        pltpu.make_async_copy(rows[j % _D], acc_sh.at[idx_v.at[j % _DI, 1]],
                              ssem[j % _D]).wait()
    plsc.subcore_barrier()
    pltpu.sync_copy(acc_sh.at[pl.ds(s * SLAB, SLAB)],
                    out_hbm.at[c, pl.ds(s * SLAB, SLAB)])


def _mm_scale_body(x_ref, wg_ref, degs_ref, hp_ref):
    h = jnp.dot(x_ref[...], wg_ref[...], preferred_element_type=jnp.float32)
    d = degs_ref[:, 0:1] + degs_ref[:, 1:2] + 1.0
    hp_ref[...] = h * lax.rsqrt(d)


def _tail_body(acc_ref, hp_ref, degs_ref, bg_ref, w1_ref, b1_ref, w2_ref,
               b2_ref, y_ref):
    agg = acc_ref[0] + acc_ref[1] + hp_ref[...]
    dis = lax.rsqrt(degs_ref[:, 0:1] + degs_ref[:, 1:2] + 1.0)
    m = jnp.maximum(agg * dis + bg_ref[...], 0.0)
    h2 = jnp.maximum(
        jnp.dot(m, w1_ref[...], preferred_element_type=jnp.float32)
        + b1_ref[...], 0.0)
    y_ref[...] = (jnp.dot(h2, w2_ref[...], preferred_element_type=jnp.float32)
                  + b2_ref[...])


def kernel(x, edge_index, Wg, bg, W1, b1, W2, b2):
    N, F = x.shape
    E = edge_index.shape[1]
    K = -(-E // (NW * L))
    EP = NW * K * L
    NP = ((N + NS * 8 - 1) // (NS * 8)) * (NS * 8) + NS * 8
    SLAB = NP // NS

    src = edge_index[0].astype(jnp.int32)
    dst = edge_index[1].astype(jnp.int32)
    pad = EP - E
    if pad:
        src = jnp.concatenate([src, jnp.zeros((pad,), jnp.int32)])
        dst = jnp.concatenate([dst, jnp.full((pad,), N, jnp.int32)])
    idx = jnp.stack([src.reshape(NW, K, L), dst.reshape(NW, K, L)], axis=2)
    zeros1 = jnp.zeros((SLAB,), jnp.float32)
    zeros2 = jnp.zeros((SLAB, F), jnp.float32)

    mesh = plsc.VectorSubcoreMesh(core_axis_name="c", subcore_axis_name="s")

    deg_call = pl.kernel(
        functools.partial(_deg_body, K=K, SLAB=SLAB),
        out_type=jax.ShapeDtypeStruct((NC, NP), jnp.float32),
        mesh=mesh,
        scratch_types=[
            pltpu.VMEM((K, 2, L), jnp.int32),
            pltpu.VMEM((L,), jnp.float32),
            pltpu.VMEM_SHARED((NP,), jnp.float32),
        ],
    )
    degs = deg_call(idx, zeros1)
    degs_t = degs.T

    RB = 1000
    grid = N // RB
    hp = pl.pallas_call(
        _mm_scale_body,
        grid=(grid,),
        in_specs=[
            pl.BlockSpec((RB, F), lambda i: (i, 0)),
            pl.BlockSpec((F, F), lambda i: (0, 0)),
            pl.BlockSpec((RB, NC), lambda i: (i, 0)),
        ],
        out_specs=pl.BlockSpec((RB, F), lambda i: (i, 0)),
        out_shape=jax.ShapeDtypeStruct((N, F), jnp.float32),
    )(x, Wg, degs_t)

    edge_call = pl.kernel(
        functools.partial(_edge_body, K=K, SLAB=SLAB),
        out_type=jax.ShapeDtypeStruct((NC, NP, F), jnp.float32),
        mesh=mesh,
        scratch_types=(
            [pltpu.VMEM((_DI, 2, L), jnp.int32)]
            + [pltpu.VMEM((L, F), jnp.float32)] * _D
            + [pltpu.VMEM_SHARED((NP, F), jnp.float32)]
            + [pltpu.SemaphoreType.DMA] * (_DI + 2 * _D)
        ),
    )
    acc = edge_call(hp, idx, zeros2)

    y = pl.pallas_call(
        _tail_body,
        grid=(grid,),
        in_specs=[
            pl.BlockSpec((NC, RB, F), lambda i: (0, i, 0)),
            pl.BlockSpec((RB, F), lambda i: (i, 0)),
            pl.BlockSpec((RB, NC), lambda i: (i, 0)),
            pl.BlockSpec((1, F), lambda i: (0, 0)),
            pl.BlockSpec((F, F), lambda i: (0, 0)),
            pl.BlockSpec((1, F), lambda i: (0, 0)),
            pl.BlockSpec((F, 1), lambda i: (0, 0)),
            pl.BlockSpec((1, 1), lambda i: (0, 0)),
        ],
        out_specs=pl.BlockSpec((RB, 1), lambda i: (i, 0)),
        out_shape=jax.ShapeDtypeStruct((N, 1), jnp.float32),
    )(acc, hp, degs_t, bg.reshape(1, F), W1, b1.reshape(1, F),
      W2, b2.reshape(1, 1))
    return y


# speedup vs baseline: 40.9655x; 1.0392x over previous
import functools

import jax
import jax.numpy as jnp
from jax import lax
from jax.experimental import pallas as pl
from jax.experimental.pallas import tpu as pltpu
from jax.experimental.pallas import tpu_sc as plsc

NC = 2
NS = 16
NW = NC * NS
L = 80
_D = 4
_DI = 8
_AI = 4
_AG = 2
_S = 2


_SD = 4


def _deg_body(idx_hbm, zeros_hbm, out_hbm, dst_v, ones_v, deg_sh,
              d0, d1, d2, d3, K, SLAB):
    c = lax.axis_index("c")
    s = lax.axis_index("s")
    w = c * NS + s
    dsem = (d0, d1, d2, d3)
    pltpu.sync_copy(zeros_hbm, deg_sh.at[pl.ds(s * SLAB, SLAB)])
    pltpu.sync_copy(idx_hbm.at[w], dst_v)
    for k in range(L // 16):
        ones_v[pl.ds(k * 16, 16)] = jnp.ones((16,), jnp.float32)
    plsc.subcore_barrier()

    def body(j, carry):
        for u in range(_SD):

            @pl.when(j % _SD == u)
            def _(u=u):
                @pl.when(j >= _SD)
                def _():
                    pltpu.make_async_copy(
                        ones_v, deg_sh.at[dst_v.at[j - _SD, 1]],
                        dsem[u]).wait()

                pltpu.async_copy(ones_v, deg_sh.at[dst_v.at[j, 1]], dsem[u],
                                 add=True)

        return carry

    lax.fori_loop(0, K, body, 0)
    for j in range(max(K - _SD, 0), K):
        pltpu.make_async_copy(ones_v, deg_sh.at[dst_v.at[j, 1]],
                              dsem[j % _SD]).wait()
    plsc.subcore_barrier()
    pltpu.sync_copy(deg_sh.at[pl.ds(s * SLAB, SLAB)],
                    out_hbm.at[c, pl.ds(s * SLAB, SLAB)])


def _edge_body(hp_hbm, idx_hbm, zeros_hbm, out_hbm,
               idx_v, r0, r1, r2, r3, acc_sh,
               i0, i1, i2, i3, i4, i5, i6, i7,
               g0, g1, g2, g3, s0, s1, s2, s3, K, SLAB):
    c = lax.axis_index("c")
    s = lax.axis_index("s")
    w = c * NS + s
    rows = (r0, r1, r2, r3)
    isem = (i0, i1, i2, i3, i4, i5, i6, i7)
    gsem = (g0, g1, g2, g3)
    ssem = (s0, s1, s2, s3)
    pltpu.sync_copy(zeros_hbm, acc_sh.at[pl.ds(s * SLAB, SLAB)])
    plsc.subcore_barrier()

    for j in range(min(_AI, K)):
        pltpu.async_copy(idx_hbm.at[w, j], idx_v.at[j % _DI], isem[j % _DI])
    for j in range(min(_AG, K)):
        si = j % _DI
        pltpu.make_async_copy(idx_hbm.at[w, j], idx_v.at[si], isem[si]).wait()
        pltpu.async_copy(hp_hbm.at[idx_v.at[si, 0]], rows[j % _D],
                         gsem[j % _D])

    def body(j, carry):
        for u in range(8):

            @pl.when(j % 8 == u)
            def _(u=u):
                p = u % _D

                @pl.when(j >= _S)
                def _():
                    q = (u - _S) % _D
                    qi = (u - _S) % _DI
                    pltpu.make_async_copy(
                        rows[q], acc_sh.at[idx_v.at[qi, 1]], ssem[q]).wait()

                @pl.when(j + _AI <= K - 1)
                def _():
                    si = (u + _AI) % _DI
                    pltpu.async_copy(idx_hbm.at[w, j + _AI], idx_v.at[si],
                                     isem[si])

                @pl.when(j + _AG <= K - 1)
                def _():
                    sg = (u + _AG) % _DI
                    rq = (u + _AG) % _D
                    pltpu.make_async_copy(idx_hbm.at[w, j + _AG],
                                          idx_v.at[sg], isem[sg]).wait()
                    pltpu.async_copy(hp_hbm.at[idx_v.at[sg, 0]], rows[rq],
                                     gsem[rq])

                ui = u % _DI
                pltpu.make_async_copy(hp_hbm.at[idx_v.at[ui, 0]], rows[p],
                                      gsem[p]).wait()
                pltpu.async_copy(rows[p], acc_sh.at[idx_v.at[ui, 1]], ssem[p],
                                 add=True)

        return carry

    lax.fori_loop(0, K, body, 0)
    for j in range(max(K - _S, 0), K):
        pltpu.make_async_copy(rows[j % _D], acc_sh.at[idx_v.at[j % _DI, 1]],
                              ssem[j % _D]).wait()
    plsc.subcore_barrier()
    pltpu.sync_copy(acc_sh.at[pl.ds(s * SLAB, SLAB)],
                    out_hbm.at[c, pl.ds(s * SLAB, SLAB)])


def _mm_scale_body(x_ref, wg_ref, degs_ref, hp_ref):
    h = jnp.dot(x_ref[...], wg_ref[...], preferred_element_type=jnp.float32)
    d = degs_ref[:, 0:1] + degs_ref[:, 1:2] + 1.0
    hp_ref[...] = h * lax.rsqrt(d)


def _tail_body(acc_ref, hp_ref, degs_ref, bg_ref, w1_ref, b1_ref, w2_ref,
               b2_ref, y_ref):
    agg = acc_ref[0] + acc_ref[1] + hp_ref[...]
    dis = lax.rsqrt(degs_ref[:, 0:1] + degs_ref[:, 1:2] + 1.0)
    m = jnp.maximum(agg * dis + bg_ref[...], 0.0)
    h2 = jnp.maximum(
        jnp.dot(m, w1_ref[...], preferred_element_type=jnp.float32)
        + b1_ref[...], 0.0)
    y_ref[...] = (jnp.dot(h2, w2_ref[...], preferred_element_type=jnp.float32)
                  + b2_ref[...])


def kernel(x, edge_index, Wg, bg, W1, b1, W2, b2):
    N, F = x.shape
    E = edge_index.shape[1]
    K = -(-E // (NW * L))
    EP = NW * K * L
    NP = ((N + NS * 8 - 1) // (NS * 8)) * (NS * 8) + NS * 8
    SLAB = NP // NS

    src = edge_index[0].astype(jnp.int32)
    dst = edge_index[1].astype(jnp.int32)
    pad = EP - E
    if pad:
        src = jnp.concatenate([src, jnp.zeros((pad,), jnp.int32)])
        dst = jnp.concatenate([dst, jnp.full((pad,), N, jnp.int32)])
    idx = jnp.stack([src.reshape(NW, K, L), dst.reshape(NW, K, L)], axis=2)
    zeros1 = jnp.zeros((SLAB,), jnp.float32)
    zeros2 = jnp.zeros((SLAB, F), jnp.float32)

    mesh = plsc.VectorSubcoreMesh(core_axis_name="c", subcore_axis_name="s")

    deg_call = pl.kernel(
        functools.partial(_deg_body, K=K, SLAB=SLAB),
        out_type=jax.ShapeDtypeStruct((NC, NP), jnp.float32),
        mesh=mesh,
        scratch_types=(
            [pltpu.VMEM((K, 2, L), jnp.int32),
             pltpu.VMEM((L,), jnp.float32),
             pltpu.VMEM_SHARED((NP,), jnp.float32)]
            + [pltpu.SemaphoreType.DMA] * _SD
        ),
    )
    degs = deg_call(idx, zeros1)
    degs_t = degs.T

    RB = 1000
    grid = N // RB
    hp = pl.pallas_call(
        _mm_scale_body,
        grid=(grid,),
        in_specs=[
            pl.BlockSpec((RB, F), lambda i: (i, 0)),
            pl.BlockSpec((F, F), lambda i: (0, 0)),
            pl.BlockSpec((RB, NC), lambda i: (i, 0)),
        ],
        out_specs=pl.BlockSpec((RB, F), lambda i: (i, 0)),
        out_shape=jax.ShapeDtypeStruct((N, F), jnp.float32),
    )(x, Wg, degs_t)

    edge_call = pl.kernel(
        functools.partial(_edge_body, K=K, SLAB=SLAB),
        out_type=jax.ShapeDtypeStruct((NC, NP, F), jnp.float32),
        mesh=mesh,
        scratch_types=(
            [pltpu.VMEM((_DI, 2, L), jnp.int32)]
            + [pltpu.VMEM((L, F), jnp.float32)] * _D
            + [pltpu.VMEM_SHARED((NP, F), jnp.float32)]
            + [pltpu.SemaphoreType.DMA] * (_DI + 2 * _D)
        ),
    )
    acc = edge_call(hp, idx, zeros2)

    y = pl.pallas_call(
        _tail_body,
        grid=(grid,),
        in_specs=[
            pl.BlockSpec((NC, RB, F), lambda i: (0, i, 0)),
            pl.BlockSpec((RB, F), lambda i: (i, 0)),
            pl.BlockSpec((RB, NC), lambda i: (i, 0)),
            pl.BlockSpec((1, F), lambda i: (0, 0)),
            pl.BlockSpec((F, F), lambda i: (0, 0)),
            pl.BlockSpec((1, F), lambda i: (0, 0)),
            pl.BlockSpec((F, 1), lambda i: (0, 0)),
            pl.BlockSpec((1, 1), lambda i: (0, 0)),
        ],
        out_specs=pl.BlockSpec((RB, 1), lambda i: (i, 0)),
        out_shape=jax.ShapeDtypeStruct((N, 1), jnp.float32),
    )(acc, hp, degs_t, bg.reshape(1, F), W1, b1.reshape(1, F),
      W2, b2.reshape(1, 1))
    return y
